# Initial kernel scaffold; baseline (speedup 1.0000x reference)
import jax
import jax.numpy as jnp
from jax.experimental import pallas as pl


def kernel(input_ids, table):
    raise NotImplementedError("write your pallas kernel here")


# speedup vs baseline: 1.5543x; 1.5543x over previous
import jax
import jax.numpy as jnp
from jax import lax
from jax.experimental import pallas as pl
from jax.experimental.pallas import tpu as pltpu
from jax.experimental.pallas import tpu_sc as plsc

EMBED_DIM = 32

_info = plsc.get_sparse_core_info()
_NC, _NS = _info.num_cores, _info.num_subcores
_NW = _NC * _NS

_CHUNK = 1600


def _make_gather(B: int, V: int, D: int):
    assert B % _NW == 0
    b_per_w = B // _NW
    assert b_per_w % _CHUNK == 0
    n_chunks = b_per_w // _CHUNK
    mesh = plsc.VectorSubcoreMesh(core_axis_name="c", subcore_axis_name="s")

    def body(idx_hbm, table_hbm, out_hbm, idx_v, rows_v, sem):
        wid = lax.axis_index("s") * _NC + lax.axis_index("c")
        base = wid * b_per_w

        def step(i, carry):
            start = base + i * _CHUNK
            pltpu.sync_copy(idx_hbm.at[pl.ds(start, _CHUNK)], idx_v)
            pltpu.async_copy(table_hbm.at[idx_v], rows_v, sem).wait()
            pltpu.sync_copy(rows_v, out_hbm.at[pl.ds(start, _CHUNK)])
            return carry

        lax.fori_loop(0, n_chunks, step, 0, unroll=False)

    return pl.kernel(
        body,
        out_type=jax.ShapeDtypeStruct((B, D), jnp.float32),
        mesh=mesh,
        scratch_types=[
            pltpu.VMEM((_CHUNK,), jnp.int32),
            pltpu.VMEM((_CHUNK, D), jnp.float32),
            pltpu.SemaphoreType.DMA,
        ],
        compiler_params=pltpu.CompilerParams(use_tc_tiling_on_sc=False),
    )


def kernel(input_ids, table):
    Bt, T = input_ids.shape
    V, D = table.shape
    flat_ids = input_ids.reshape(-1).astype(jnp.int32)
    out = _make_gather(flat_ids.shape[0], V, D)(flat_ids, table)
    return out.reshape(Bt, T, D)


# speedup vs baseline: 1.5708x; 1.0106x over previous
import jax
import jax.numpy as jnp
from jax import lax
from jax.experimental import pallas as pl
from jax.experimental.pallas import tpu as pltpu
from jax.experimental.pallas import tpu_sc as plsc

EMBED_DIM = 32

_info = plsc.get_sparse_core_info()
_NC, _NS = _info.num_cores, _info.num_subcores
_NW = _NC * _NS

_CHUNK = 1600


def _make_gather(B: int, V: int, D: int):
    assert B % _NW == 0
    b_per_w = B // _NW
    assert b_per_w % _CHUNK == 0
    n_chunks = b_per_w // _CHUNK
    mesh = plsc.VectorSubcoreMesh(core_axis_name="c", subcore_axis_name="s")

    def body(idx_hbm, table_hbm, out_hbm, idx_v, rows_v, sem_idx, sem_gat,
             sem_out):
        wid = lax.axis_index("s") * _NC + lax.axis_index("c")
        base = wid * b_per_w

        def idx_copy(i):
            return pltpu.make_async_copy(
                idx_hbm.at[pl.ds(base + i * _CHUNK, _CHUNK)],
                idx_v.at[i % 2], sem_idx)

        def gat_copy(i):
            return pltpu.make_async_copy(
                table_hbm.at[idx_v.at[i % 2]], rows_v.at[i % 2], sem_gat)

        def out_copy(i):
            return pltpu.make_async_copy(
                rows_v.at[i % 2],
                out_hbm.at[pl.ds(base + i * _CHUNK, _CHUNK)], sem_out)

        idx_copy(0).start()
        if n_chunks > 1:
            idx_copy(1).start()
        idx_copy(0).wait()
        gat_copy(0).start()
        for i in range(n_chunks):
            gat_copy(i).wait()
            out_copy(i).start()
            if i + 2 < n_chunks:
                idx_copy(i + 2).start()
            if i + 1 < n_chunks:
                idx_copy(i + 1).wait()
                if i >= 1:
                    out_copy(i - 1).wait()
                gat_copy(i + 1).start()
        if n_chunks >= 2:
            out_copy(n_chunks - 2).wait()
        out_copy(n_chunks - 1).wait()

    return pl.kernel(
        body,
        out_type=jax.ShapeDtypeStruct((B, D), jnp.float32),
        mesh=mesh,
        scratch_types=[
            pltpu.VMEM((2, _CHUNK), jnp.int32),
            pltpu.VMEM((2, _CHUNK, D), jnp.float32),
            pltpu.SemaphoreType.DMA,
            pltpu.SemaphoreType.DMA,
            pltpu.SemaphoreType.DMA,
        ],
        compiler_params=pltpu.CompilerParams(use_tc_tiling_on_sc=False),
    )


def kernel(input_ids, table):
    Bt, T = input_ids.shape
    V, D = table.shape
    flat_ids = input_ids.reshape(-1).astype(jnp.int32)
    out = _make_gather(flat_ids.shape[0], V, D)(flat_ids, table)
    return out.reshape(Bt, T, D)
